# baseline (speedup 1.0000x reference)
import numpy as np

import jax
import jax.numpy as jnp
from jax.experimental import pallas as pl

_TAU = 1.0
_B, _N, _M = 64, 512, 512
_NM = _N * _M
_FBB = 8


def _fused_body(x_ref, o_ref, ir_ref, ic_ref, out_ref):
    e = jnp.exp(-x_ref[...] * (1.0 / _TAU)) * o_ref[...]
    big = jnp.int32(2**31 - 1)
    for j in range(_FBB):
        ej = e[j : j + 1]
        m = jnp.max(ej)
        rid = jnp.min(jnp.where(ej == m, ic_ref[...], big), axis=1)
        flat = jnp.min(jnp.where(rid < big, rid * _M + ir_ref[0], big))
        r = flat // _M
        c = flat - r * _M
        onehot = (ic_ref[...] == r) & (ir_ref[...] == c)
        out_ref[pl.ds(j, 1)] = onehot.astype(jnp.float32)


def kernel(x, o):
    ir = np.arange(_M, dtype=np.int32).reshape(1, 1, _M)
    ic = np.arange(_N, dtype=np.int32).reshape(1, _N, 1)
    return pl.pallas_call(
        _fused_body,
        grid=(_B // _FBB,),
        in_specs=[
            pl.BlockSpec((_FBB, _N, _M), lambda b: (b, 0, 0)),
            pl.BlockSpec((_FBB, _N, _M), lambda b: (b, 0, 0)),
            pl.BlockSpec((1, 1, _M), lambda b: (0, 0, 0)),
            pl.BlockSpec((1, _N, 1), lambda b: (0, 0, 0)),
        ],
        out_specs=pl.BlockSpec((_FBB, _N, _M), lambda b: (b, 0, 0)),
        out_shape=jax.ShapeDtypeStruct((_B, _N, _M), jnp.float32),
    )(x, o, ir, ic)
